# Initial kernel scaffold; baseline (speedup 1.0000x reference)
import jax
import jax.numpy as jnp
from jax.experimental import pallas as pl


def kernel(hidden_states, W_down, b_down, rms_w, W1, b1, W2, b2, W3, balancing_biases):
    raise NotImplementedError("write your pallas kernel here")


# speedup vs baseline: 1.0583x; 1.0583x over previous
import jax
import jax.numpy as jnp
from jax.experimental import pallas as pl
from jax.experimental.pallas import tpu as pltpu

S, H, D, E, K = 16384, 4096, 2048, 64, 8
EPS = 1e-05
PREC = jax.lax.Precision.DEFAULT

BM1 = 256
BM2 = 256


_F = jnp.float32


def _gelu(xv):
    u = (-xv) * _F(0.707106769)
    z2 = u * u
    ax = jnp.abs(u)
    p = _F(7.85386146e-05)
    for c in (-0.000801019371, 0.00518832775, -0.0268538129, 0.112835854,
              -0.37612626, 1.12837911):
        p = p * z2 + _F(c)
    res_small = _F(1.0) - u * p
    nz2 = -z2
    e = jnp.exp(nz2)
    q = e * (_F(1.0) / ax)
    w = _F(1.0) / z2
    r1 = w * _F(0.0232682)
    for c in (-0.138703942, 0.368742466, -0.582473278, 0.621000469,
              -0.494451523, 0.340488, -0.274112701):
        r1 = (r1 + _F(c)) * w
    r1 = r1 + _F(0.563825965)
    r2 = w * _F(-10.477664)
    for c in (12.9772, -7.49551868, 2.92101908, -1.01526523, 0.42184633,
              -0.282076746):
        r2 = (r2 + _F(c)) * w
    r2 = r2 + _F(0.564189494)
    sel = jnp.where(ax < _F(2.0), r1, r2)
    t = q * sel
    t = jnp.where(nz2 < _F(-88.7228394), _F(0.0), t)
    res_large = jnp.where(u < _F(0.0), _F(2.0) - t, t)
    out = jnp.where(ax < _F(1.0), res_small, res_large)
    return (xv * _F(0.5)) * out


def _foldhalf(a):
    n = a.shape[1]
    while n > 1:
        a = a[:, :n // 2] + a[:, n // 2:n]
        n //= 2
    return a


def _down_kernel(x_ref, w_ref, b_ref, o_ref):
    o_ref[...] = jnp.dot(x_ref[...], w_ref[...],
                         preferred_element_type=jnp.float32,
                         precision=PREC) + b_ref[...]


def _router_kernel(hs_ref, var_ref, rms_ref, w1_ref, b1_ref, w2_ref, b2_ref,
                   w3_ref, be_ref, rp_ref, idx_ref):
    hs = hs_ref[...]
    var = var_ref[...]
    hsn = hs * jax.lax.rsqrt(var + EPS) * rms_ref[...]
    h = jnp.dot(hsn, w1_ref[...], preferred_element_type=jnp.float32,
                precision=PREC) + b1_ref[...]
    h = _gelu(h)
    h = jnp.dot(h, w2_ref[...], preferred_element_type=jnp.float32,
                precision=PREC) + b2_ref[...]
    h = _gelu(h)
    logits = jnp.dot(h, w3_ref[...], preferred_element_type=jnp.float32,
                     precision=PREC)
    m = jnp.max(logits, axis=-1, keepdims=True)
    ex = jnp.exp(logits - m)
    p = ex / _foldhalf(ex)
    vals = p + be_ref[...]
    iota = jax.lax.broadcasted_iota(jnp.int32, vals.shape, 1)
    rp_cols, idx_cols = [], []
    for _ in range(K):
        mx = jnp.max(vals, axis=-1, keepdims=True)
        idx = jnp.min(jnp.where(vals == mx, iota, E), axis=-1, keepdims=True)
        sel = iota == idx
        rp_cols.append(jnp.sum(jnp.where(sel, p, 0.0), axis=-1, keepdims=True))
        idx_cols.append(idx)
        vals = jnp.where(sel, -jnp.inf, vals)
    rp_ref[...] = jnp.concatenate(rp_cols, axis=1)
    idx_ref[...] = jnp.concatenate(idx_cols, axis=1)


def kernel(hidden_states, W_down, b_down, rms_w, W1, b1, W2, b2, W3,
           balancing_biases):
    hs = pl.pallas_call(
        _down_kernel,
        grid=(S // BM1,),
        in_specs=[
            pl.BlockSpec((BM1, H), lambda i: (i, 0)),
            pl.BlockSpec((H, D), lambda i: (0, 0)),
            pl.BlockSpec((1, D), lambda i: (0, 0)),
        ],
        out_specs=pl.BlockSpec((BM1, D), lambda i: (i, 0)),
        out_shape=jax.ShapeDtypeStruct((S, D), jnp.float32),
        compiler_params=pltpu.CompilerParams(
            dimension_semantics=("arbitrary",)),
    )(hidden_states, W_down, b_down.reshape(1, D))

    var = jnp.mean(hs * hs, axis=-1, keepdims=True)

    rp, idx = pl.pallas_call(
        _router_kernel,
        grid=(S // BM2,),
        in_specs=[
            pl.BlockSpec((BM2, D), lambda i: (i, 0)),
            pl.BlockSpec((BM2, 1), lambda i: (i, 0)),
            pl.BlockSpec((1, D), lambda i: (0, 0)),
            pl.BlockSpec((D, D), lambda i: (0, 0)),
            pl.BlockSpec((1, D), lambda i: (0, 0)),
            pl.BlockSpec((D, D), lambda i: (0, 0)),
            pl.BlockSpec((1, D), lambda i: (0, 0)),
            pl.BlockSpec((D, E), lambda i: (0, 0)),
            pl.BlockSpec((1, E), lambda i: (0, 0)),
        ],
        out_specs=[
            pl.BlockSpec((BM2, K), lambda i: (i, 0)),
            pl.BlockSpec((BM2, K), lambda i: (i, 0)),
        ],
        out_shape=[
            jax.ShapeDtypeStruct((S, K), jnp.float32),
            jax.ShapeDtypeStruct((S, K), jnp.int32),
        ],
        compiler_params=pltpu.CompilerParams(
            dimension_semantics=("arbitrary",)),
    )(hs, var, rms_w.reshape(1, D), W1, b1.reshape(1, D), W2,
      b2.reshape(1, D), W3, balancing_biases.reshape(1, E))

    return rp, idx.astype(jnp.int64), hs
